# baseline (speedup 1.0000x reference)
import jax
import jax.numpy as jnp
from jax.experimental import pallas as pl
from jax.experimental.pallas import tpu as pltpu

TOKEN_BLOCK = 512
DEPTH = 6


def _router_kernel(x_hbm, w_ref, b_ref, w_out_ref, l_out_ref, buf, sems):
    i = pl.program_id(0)
    nsteps = pl.num_programs(0)

    def start_copy(slot, chunk):
        pltpu.make_async_copy(
            x_hbm.at[pl.ds(chunk * TOKEN_BLOCK, TOKEN_BLOCK), :],
            buf.at[slot],
            sems.at[slot],
        ).start()

    @pl.when(i == 0)
    def _prologue():
        for s in range(DEPTH):
            start_copy(s, s)

    slot = jax.lax.rem(i, DEPTH)
    pltpu.make_async_copy(
        x_hbm.at[pl.ds(i * TOKEN_BLOCK, TOKEN_BLOCK), :],
        buf.at[slot],
        sems.at[slot],
    ).wait()

    logits = jax.lax.dot_general(
        buf[slot], w_ref[...],
        dimension_numbers=(((1,), (1,)), ((), ())),
        preferred_element_type=jnp.float32,
    ) + b_ref[...]
    l_out_ref[...] = logits
    m = jnp.max(logits, axis=1, keepdims=True)
    e = jnp.exp(logits - m)
    s = jnp.sum(e, axis=1, keepdims=True)
    w_out_ref[...] = e / s

    @pl.when(i + DEPTH < nsteps)
    def _refill():
        start_copy(slot, i + DEPTH)


def kernel(x, W, b):
    tokens, feat = x.shape
    n_exp = W.shape[0]
    b2 = b.reshape(1, n_exp)
    nsteps = tokens // TOKEN_BLOCK
    weights, logits = pl.pallas_call(
        _router_kernel,
        grid=(nsteps,),
        in_specs=[
            pl.BlockSpec(memory_space=pltpu.MemorySpace.HBM),
            pl.BlockSpec((n_exp, feat), lambda i: (0, 0)),
            pl.BlockSpec((1, n_exp), lambda i: (0, 0)),
        ],
        out_specs=[
            pl.BlockSpec((TOKEN_BLOCK, n_exp), lambda i: (i, 0)),
            pl.BlockSpec((TOKEN_BLOCK, n_exp), lambda i: (i, 0)),
        ],
        out_shape=[
            jax.ShapeDtypeStruct((tokens, n_exp), jnp.float32),
            jax.ShapeDtypeStruct((tokens, n_exp), jnp.float32),
        ],
        scratch_shapes=[
            pltpu.VMEM((DEPTH, TOKEN_BLOCK, feat), jnp.float32),
            pltpu.SemaphoreType.DMA((DEPTH,)),
        ],
        compiler_params=pltpu.CompilerParams(
            skip_device_barrier=True,
            disable_bounds_checks=True,
            disable_semaphore_checks=True,
        ),
    )(x, W, b2)
    return (weights, logits)


# speedup vs baseline: 1.0034x; 1.0034x over previous
import jax
import jax.numpy as jnp
from jax.experimental import pallas as pl
from jax.experimental.pallas import tpu as pltpu

TOKEN_BLOCK = 256
DEPTH = 8
OUT_GROUP = 8


def _router_kernel(x_hbm, w_ref, b_ref, w_out_ref, l_out_ref, buf, sems):
    i = pl.program_id(0)
    nsteps = pl.num_programs(0)

    def start_copy(slot, chunk):
        pltpu.make_async_copy(
            x_hbm.at[pl.ds(chunk * TOKEN_BLOCK, TOKEN_BLOCK), :],
            buf.at[slot],
            sems.at[slot],
        ).start()

    @pl.when(i == 0)
    def _prologue():
        for s in range(DEPTH):
            start_copy(s, s)

    slot = jax.lax.rem(i, DEPTH)
    pltpu.make_async_copy(
        x_hbm.at[pl.ds(i * TOKEN_BLOCK, TOKEN_BLOCK), :],
        buf.at[slot],
        sems.at[slot],
    ).wait()

    logits = jax.lax.dot_general(
        buf[slot], w_ref[...],
        dimension_numbers=(((1,), (1,)), ((), ())),
        preferred_element_type=jnp.float32,
    ) + b_ref[...]
    row = jax.lax.rem(i, OUT_GROUP) * TOKEN_BLOCK
    l_out_ref[pl.ds(row, TOKEN_BLOCK), :] = logits
    m = jnp.max(logits, axis=1, keepdims=True)
    e = jnp.exp(logits - m)
    s = jnp.sum(e, axis=1, keepdims=True)
    w_out_ref[pl.ds(row, TOKEN_BLOCK), :] = e / s

    @pl.when(i + DEPTH < nsteps)
    def _refill():
        start_copy(slot, i + DEPTH)


def kernel(x, W, b):
    tokens, feat = x.shape
    n_exp = W.shape[0]
    b2 = b.reshape(1, n_exp)
    nsteps = tokens // TOKEN_BLOCK
    weights, logits = pl.pallas_call(
        _router_kernel,
        grid=(nsteps,),
        in_specs=[
            pl.BlockSpec(memory_space=pltpu.MemorySpace.HBM),
            pl.BlockSpec((n_exp, feat), lambda i: (0, 0)),
            pl.BlockSpec((1, n_exp), lambda i: (0, 0)),
        ],
        out_specs=[
            pl.BlockSpec((OUT_GROUP * TOKEN_BLOCK, n_exp), lambda i: (i // OUT_GROUP, 0)),
            pl.BlockSpec((OUT_GROUP * TOKEN_BLOCK, n_exp), lambda i: (i // OUT_GROUP, 0)),
        ],
        out_shape=[
            jax.ShapeDtypeStruct((tokens, n_exp), jnp.float32),
            jax.ShapeDtypeStruct((tokens, n_exp), jnp.float32),
        ],
        scratch_shapes=[
            pltpu.VMEM((DEPTH, TOKEN_BLOCK, feat), jnp.float32),
            pltpu.SemaphoreType.DMA((DEPTH,)),
        ],
        compiler_params=pltpu.CompilerParams(
            skip_device_barrier=True,
            disable_bounds_checks=True,
            disable_semaphore_checks=True,
        ),
    )(x, W, b2)
    return (weights, logits)


# speedup vs baseline: 1.0102x; 1.0068x over previous
import jax
import jax.numpy as jnp
from jax.experimental import pallas as pl
from jax.experimental.pallas import tpu as pltpu

TOKEN_BLOCK = 256
DEPTH = 8


def _router_kernel(x_hbm, w_ref, b_ref, w_out_ref, l_out_ref, buf, sems):
    i = pl.program_id(0)
    nsteps = pl.num_programs(0)

    def start_copy(slot, chunk):
        pltpu.make_async_copy(
            x_hbm.at[pl.ds(chunk * TOKEN_BLOCK, TOKEN_BLOCK), :],
            buf.at[slot],
            sems.at[slot],
        ).start()

    @pl.when(i == 0)
    def _prologue():
        for s in range(DEPTH):
            start_copy(s, s)

    slot = jax.lax.rem(i, DEPTH)
    pltpu.make_async_copy(
        x_hbm.at[pl.ds(i * TOKEN_BLOCK, TOKEN_BLOCK), :],
        buf.at[slot],
        sems.at[slot],
    ).wait()

    logits = jax.lax.dot_general(
        buf[slot], w_ref[...],
        dimension_numbers=(((1,), (1,)), ((), ())),
        preferred_element_type=jnp.float32,
    ) + b_ref[...]
    l_out_ref[...] = logits
    m = jnp.max(logits, axis=1, keepdims=True)
    e = jnp.exp(logits - m)
    s = jnp.sum(e, axis=1, keepdims=True)
    w_out_ref[...] = e / s

    @pl.when(i + DEPTH < nsteps)
    def _refill():
        start_copy(slot, i + DEPTH)


def kernel(x, W, b):
    tokens, feat = x.shape
    n_exp = W.shape[0]
    b2 = b.reshape(1, n_exp)
    nsteps = tokens // TOKEN_BLOCK
    weights, logits = pl.pallas_call(
        _router_kernel,
        grid=(nsteps,),
        in_specs=[
            pl.BlockSpec(memory_space=pltpu.MemorySpace.HBM),
            pl.BlockSpec((n_exp, feat), lambda i: (0, 0)),
            pl.BlockSpec((1, n_exp), lambda i: (0, 0)),
        ],
        out_specs=[
            pl.BlockSpec((TOKEN_BLOCK, n_exp), lambda i: (i, 0)),
            pl.BlockSpec((TOKEN_BLOCK, n_exp), lambda i: (i, 0)),
        ],
        out_shape=[
            jax.ShapeDtypeStruct((tokens, n_exp), jnp.float32),
            jax.ShapeDtypeStruct((tokens, n_exp), jnp.float32),
        ],
        scratch_shapes=[
            pltpu.VMEM((DEPTH, TOKEN_BLOCK, feat), jnp.float32),
            pltpu.SemaphoreType.DMA((DEPTH,)),
        ],
    )(x, W, b2)
    return (weights, logits)
